# baseline (speedup 1.0000x reference)
import numpy as np
import jax
import jax.numpy as jnp
from jax import lax
from jax.experimental import pallas as pl
from jax.experimental.pallas import tpu as pltpu
from jax.experimental.pallas import tpu_sc as plsc

_BATCH = 64
_H = 32
_W = 32
_SEQ = _H * _W
_NB = 4
_TOTAL = _BATCH * _NB
_SCALE_MIN = 0.15
_SCALE_MAX = 0.2
_ASPECT = 0.75

_NC = 1
_NS = 16
_NW = _NC * _NS
_BPW = _BATCH // _NW


def _sqrt_thresholds(kmax: int = 40) -> np.ndarray:
    ts = []
    for k in range(1, kmax + 1):
        z = np.float32(k * k)
        while True:
            z2 = np.nextafter(z, np.float32(0), dtype=np.float32)
            if np.sqrt(z2, dtype=np.float32) >= np.float32(k):
                z = z2
            else:
                break
        ts.append(z)
    return np.array(ts, dtype=np.float32)


_THRESH = _sqrt_thresholds()


def _body(us_hbm, ut_hbm, ul_hbm, tgt_hbm,
          u_s, u_t, u_l, tbuf):
    w = lax.axis_index("s") * _NC + lax.axis_index("c")
    nblk = _BPW * _NB
    base = w * nblk

    pltpu.sync_copy(us_hbm.at[pl.ds(base, nblk)], u_s.at[pl.ds(0, nblk)])
    pltpu.sync_copy(ut_hbm.at[pl.ds(base, nblk)], u_t.at[pl.ds(0, nblk)])
    pltpu.sync_copy(ul_hbm.at[pl.ds(base, nblk)], u_l.at[pl.ds(0, nblk)])

    us = u_s[...]
    ut = u_t[...]
    ul = u_l[...]

    scales = jnp.float32(_SCALE_MIN) + us * jnp.float32(_SCALE_MAX - _SCALE_MIN)
    area = (scales * jnp.float32(_SEQ)).astype(jnp.int32)
    y = area.astype(jnp.float32) / jnp.float32(_ASPECT)
    hs_raw = jnp.zeros((16,), jnp.int32)
    for k in range(1, 34):
        hs_raw = hs_raw + jnp.where(y >= jnp.float32(_THRESH[k - 1]),
                                    jnp.int32(1), jnp.int32(0))
    hs = jnp.clip(hs_raw, 1, _H)
    ws = jnp.clip(area // hs, 1, _W)
    max_tops = _H - hs + 1
    max_lefts = _W - ws + 1
    tops = (ut * max_tops.astype(jnp.float32)).astype(jnp.int32)
    lefts = (ul * max_lefts.astype(jnp.float32)).astype(jnp.int32)

    ws_u = ws.astype(jnp.uint32)
    ones_w = (jnp.uint32(2) << (ws_u - 1)) - 1
    lef_u = jnp.minimum(lefts, 31).astype(jnp.uint32)
    cm = jnp.where(lefts > 31, jnp.uint32(0), ones_w << lef_u)

    tend = tops + hs
    cm_i = plsc.bitcast(cm, jnp.int32)

    iota = lax.iota(jnp.int32, 16)
    r_lo = iota
    r_hi = iota + 16
    shiftv = ((iota & 7) << 2).astype(jnp.uint32)
    lsb4 = jnp.full((16,), 0x01010101, jnp.uint32)
    nib = jnp.full((16,), 0xF, jnp.uint32)

    def batch_body(b, carry):
        def blk_body(j, bits):
            bl, bh = bits
            idx = jnp.full((16,), 0, jnp.int32) + (_NB * b + j)
            topb = tops.at[idx].get(mode="promise_in_bounds")
            tendb = tend.at[idx].get(mode="promise_in_bounds")
            cmb = plsc.bitcast(cm_i.at[idx].get(mode="promise_in_bounds"),
                               jnp.uint32)
            z = jnp.uint32(0)
            bl = bl | jnp.where((r_lo >= topb) & (r_lo < tendb), cmb, z)
            bh = bh | jnp.where((r_hi >= topb) & (r_hi < tendb), cmb, z)
            return bl, bh

        zz = jnp.zeros((16,), jnp.uint32)
        bits_lo, bits_hi = lax.fori_loop(0, _NB, blk_body, (zz, zz))
        blo_i = plsc.bitcast(bits_lo, jnp.int32)
        bhi_i = plsc.bitcast(bits_hi, jnp.int32)

        def exp_body(t, c):
            sel = jnp.full((16,), 0, jnp.int32) + t
            src = jnp.where(sel < 8, blo_i, bhi_i)
            rowsel = (iota >> 3) + (2 * t - 16 * (t // 8))
            g = plsc.bitcast(src.at[rowsel].get(mode="promise_in_bounds"),
                             jnp.uint32)
            x = (g >> shiftv) & nib
            yb = (x | (x << 7) | (x << 14) | (x << 21)) & lsb4
            off = (_SEQ // 4) * b + 16 * t
            tbuf[pl.ds(off, 16)] = plsc.bitcast(yb, jnp.int32)
            return c

        lax.fori_loop(0, 16, exp_body, jnp.int32(0))
        return carry

    lax.fori_loop(0, _BPW, batch_body, jnp.int32(0))

    nw = _BPW * _SEQ // 4
    pltpu.sync_copy(tbuf, tgt_hbm.at[pl.ds(nw * w, nw)])


@jax.jit
def kernel(u_scale, u_top, u_left):
    mesh = plsc.VectorSubcoreMesh(core_axis_name="c", subcore_axis_name="s",
                                  num_cores=_NC, num_subcores=_NS)
    out_type = jax.ShapeDtypeStruct((_BATCH * _SEQ // 4,), jnp.int32)
    scratch = [
        pltpu.VMEM((16,), jnp.float32),
        pltpu.VMEM((16,), jnp.float32),
        pltpu.VMEM((16,), jnp.float32),
        pltpu.VMEM((_BPW * _SEQ // 4,), jnp.int32),
    ]
    run = pl.kernel(_body, out_type=out_type, mesh=mesh,
                    compiler_params=pltpu.CompilerParams(
                        needs_layout_passes=False),
                    scratch_types=scratch)
    tgtw = run(u_scale, u_top, u_left)

    cols = jnp.arange(_SEQ, dtype=jnp.int32)
    idx = (_SEQ // 4) * jnp.arange(_BATCH, dtype=jnp.int32)[:, None] \
        + (cols >> 2)[None, :]
    g = tgtw[idx]
    bits = (g >> ((cols & 3) << 3)[None, :]) & 1
    return bits == 0, bits != 0


# speedup vs baseline: 18.8915x; 18.8915x over previous
import numpy as np
import jax
import jax.numpy as jnp
from jax import lax
from jax.experimental import pallas as pl
from jax.experimental.pallas import tpu as pltpu
from jax.experimental.pallas import tpu_sc as plsc

_BATCH = 64
_H = 32
_W = 32
_SEQ = _H * _W
_NB = 4
_TOTAL = _BATCH * _NB
_SCALE_MIN = 0.15
_SCALE_MAX = 0.2
_ASPECT = 0.75

_NC = 1
_NS = 16
_NW = _NC * _NS
_BPW = _BATCH // _NW


def _sqrt_thresholds(kmax: int = 40) -> np.ndarray:
    ts = []
    for k in range(1, kmax + 1):
        z = np.float32(k * k)
        while True:
            z2 = np.nextafter(z, np.float32(0), dtype=np.float32)
            if np.sqrt(z2, dtype=np.float32) >= np.float32(k):
                z = z2
            else:
                break
        ts.append(z)
    return np.array(ts, dtype=np.float32)


_THRESH = _sqrt_thresholds()


def _body(us_hbm, ut_hbm, ul_hbm, tgt_hbm,
          u_s, u_t, u_l, tbuf):
    w = lax.axis_index("s") * _NC + lax.axis_index("c")
    nblk = _BPW * _NB
    base = w * nblk

    pltpu.sync_copy(us_hbm.at[pl.ds(base, nblk)], u_s.at[pl.ds(0, nblk)])
    pltpu.sync_copy(ut_hbm.at[pl.ds(base, nblk)], u_t.at[pl.ds(0, nblk)])
    pltpu.sync_copy(ul_hbm.at[pl.ds(base, nblk)], u_l.at[pl.ds(0, nblk)])

    us = u_s[...]
    ut = u_t[...]
    ul = u_l[...]

    scales = jnp.float32(_SCALE_MIN) + us * jnp.float32(_SCALE_MAX - _SCALE_MIN)
    area = (scales * jnp.float32(_SEQ)).astype(jnp.int32)
    y = area.astype(jnp.float32) / jnp.float32(_ASPECT)
    hs_raw = jnp.zeros((16,), jnp.int32)
    for k in range(1, 34):
        hs_raw = hs_raw + jnp.where(y >= jnp.float32(_THRESH[k - 1]),
                                    jnp.int32(1), jnp.int32(0))
    hs = jnp.clip(hs_raw, 1, _H)
    ws = jnp.clip(area // hs, 1, _W)
    max_tops = _H - hs + 1
    max_lefts = _W - ws + 1
    tops = (ut * max_tops.astype(jnp.float32)).astype(jnp.int32)
    lefts = (ul * max_lefts.astype(jnp.float32)).astype(jnp.int32)

    ws_u = ws.astype(jnp.uint32)
    ones_w = (jnp.uint32(2) << (ws_u - 1)) - 1
    lef_u = jnp.minimum(lefts, 31).astype(jnp.uint32)
    cm = jnp.where(lefts > 31, jnp.uint32(0), ones_w << lef_u)

    tend = tops + hs
    cm_i = plsc.bitcast(cm, jnp.int32)

    iota = lax.iota(jnp.int32, 16)
    r_lo = iota
    r_hi = iota + 16
    shiftv = ((iota & 7) << 2).astype(jnp.uint32)
    lsb4 = jnp.full((16,), 0x01010101, jnp.uint32)
    nib = jnp.full((16,), 0xF, jnp.uint32)

    def batch_body(b, carry):
        def blk_body(j, bits):
            bl, bh = bits
            idx = jnp.full((16,), 0, jnp.int32) + (_NB * b + j)
            topb = tops.at[idx].get(mode="promise_in_bounds")
            tendb = tend.at[idx].get(mode="promise_in_bounds")
            cmb = plsc.bitcast(cm_i.at[idx].get(mode="promise_in_bounds"),
                               jnp.uint32)
            z = jnp.uint32(0)
            bl = bl | jnp.where((r_lo >= topb) & (r_lo < tendb), cmb, z)
            bh = bh | jnp.where((r_hi >= topb) & (r_hi < tendb), cmb, z)
            return bl, bh

        zz = jnp.zeros((16,), jnp.uint32)
        bits_lo, bits_hi = lax.fori_loop(0, _NB, blk_body, (zz, zz))
        blo_i = plsc.bitcast(bits_lo, jnp.int32)
        bhi_i = plsc.bitcast(bits_hi, jnp.int32)

        def exp_body(t, c):
            sel = jnp.full((16,), 0, jnp.int32) + t
            src = jnp.where(sel < 8, blo_i, bhi_i)
            rowsel = (iota >> 3) + (2 * t - 16 * (t // 8))
            g = plsc.bitcast(src.at[rowsel].get(mode="promise_in_bounds"),
                             jnp.uint32)
            x = (g >> shiftv) & nib
            yb = (x | (x << 7) | (x << 14) | (x << 21)) & lsb4
            off = (_SEQ // 4) * b + 16 * t
            tbuf[pl.ds(off, 16)] = plsc.bitcast(yb, jnp.int32)
            return c

        lax.fori_loop(0, 16, exp_body, jnp.int32(0))
        return carry

    lax.fori_loop(0, _BPW, batch_body, jnp.int32(0))

    nw = _BPW * _SEQ // 4
    pltpu.sync_copy(tbuf, tgt_hbm.at[pl.ds(nw * w, nw)])


@jax.jit
def kernel(u_scale, u_top, u_left):
    mesh = plsc.VectorSubcoreMesh(core_axis_name="c", subcore_axis_name="s",
                                  num_cores=_NC, num_subcores=_NS)
    out_type = jax.ShapeDtypeStruct((_BATCH * _SEQ // 4,), jnp.int32)
    scratch = [
        pltpu.VMEM((16,), jnp.float32),
        pltpu.VMEM((16,), jnp.float32),
        pltpu.VMEM((16,), jnp.float32),
        pltpu.VMEM((_BPW * _SEQ // 4,), jnp.int32),
    ]
    run = pl.kernel(_body, out_type=out_type, mesh=mesh,
                    compiler_params=pltpu.CompilerParams(
                        needs_layout_passes=False),
                    scratch_types=scratch)
    tgtw = run(u_scale, u_top, u_left)

    by = lax.bitcast_convert_type(tgtw, jnp.uint8).reshape(_BATCH, _SEQ)
    return by == 0, by != 0


# speedup vs baseline: 19.5637x; 1.0356x over previous
import numpy as np
import jax
import jax.numpy as jnp
from jax import lax
from jax.experimental import pallas as pl
from jax.experimental.pallas import tpu as pltpu
from jax.experimental.pallas import tpu_sc as plsc

_BATCH = 64
_H = 32
_W = 32
_SEQ = _H * _W
_NB = 4
_TOTAL = _BATCH * _NB
_SCALE_MIN = 0.15
_SCALE_MAX = 0.2
_ASPECT = 0.75

_NC = 1
_NS = 16
_NW = _NC * _NS
_BPW = _BATCH // _NW


def _sqrt_thresholds(kmax: int = 40) -> np.ndarray:
    ts = []
    for k in range(1, kmax + 1):
        z = np.float32(k * k)
        while True:
            z2 = np.nextafter(z, np.float32(0), dtype=np.float32)
            if np.sqrt(z2, dtype=np.float32) >= np.float32(k):
                z = z2
            else:
                break
        ts.append(z)
    return np.array(ts, dtype=np.float32)


_THRESH = _sqrt_thresholds()


def _body(us_hbm, ut_hbm, ul_hbm, tgt_hbm,
          u_s, u_t, u_l, tbuf, sem):
    w = lax.axis_index("s") * _NC + lax.axis_index("c")
    nblk = _BPW * _NB
    base = w * nblk

    h1 = pltpu.make_async_copy(us_hbm.at[pl.ds(base, nblk)],
                               u_s.at[pl.ds(0, nblk)], sem)
    h2 = pltpu.make_async_copy(ut_hbm.at[pl.ds(base, nblk)],
                               u_t.at[pl.ds(0, nblk)], sem)
    h3 = pltpu.make_async_copy(ul_hbm.at[pl.ds(base, nblk)],
                               u_l.at[pl.ds(0, nblk)], sem)
    h1.start()
    h2.start()
    h3.start()
    h1.wait()
    h2.wait()
    h3.wait()

    us = u_s[...]
    ut = u_t[...]
    ul = u_l[...]

    scales = jnp.float32(_SCALE_MIN) + us * jnp.float32(_SCALE_MAX - _SCALE_MIN)
    area = (scales * jnp.float32(_SEQ)).astype(jnp.int32)
    y = area.astype(jnp.float32) / jnp.float32(_ASPECT)
    hs_raw = jnp.zeros((16,), jnp.int32)
    for k in range(1, 34):
        hs_raw = hs_raw + jnp.where(y >= jnp.float32(_THRESH[k - 1]),
                                    jnp.int32(1), jnp.int32(0))
    hs = jnp.clip(hs_raw, 1, _H)
    ws = jnp.clip(area // hs, 1, _W)
    max_tops = _H - hs + 1
    max_lefts = _W - ws + 1
    tops = (ut * max_tops.astype(jnp.float32)).astype(jnp.int32)
    lefts = (ul * max_lefts.astype(jnp.float32)).astype(jnp.int32)

    ws_u = ws.astype(jnp.uint32)
    ones_w = (jnp.uint32(2) << (ws_u - 1)) - 1
    lef_u = jnp.minimum(lefts, 31).astype(jnp.uint32)
    cm = jnp.where(lefts > 31, jnp.uint32(0), ones_w << lef_u)

    tend = tops + hs
    cm_i = plsc.bitcast(cm, jnp.int32)

    iota = lax.iota(jnp.int32, 16)
    r_lo = iota
    r_hi = iota + 16
    shiftv = ((iota & 7) << 2).astype(jnp.uint32)
    lsb4 = jnp.full((16,), 0x01010101, jnp.uint32)
    nib = jnp.full((16,), 0xF, jnp.uint32)

    def batch_body(b, carry):
        def blk_body(j, bits):
            bl, bh = bits
            idx = jnp.full((16,), 0, jnp.int32) + (_NB * b + j)
            topb = tops.at[idx].get(mode="promise_in_bounds")
            tendb = tend.at[idx].get(mode="promise_in_bounds")
            cmb = plsc.bitcast(cm_i.at[idx].get(mode="promise_in_bounds"),
                               jnp.uint32)
            z = jnp.uint32(0)
            bl = bl | jnp.where((r_lo >= topb) & (r_lo < tendb), cmb, z)
            bh = bh | jnp.where((r_hi >= topb) & (r_hi < tendb), cmb, z)
            return bl, bh

        zz = jnp.zeros((16,), jnp.uint32)
        bits_lo, bits_hi = lax.fori_loop(0, _NB, blk_body, (zz, zz))
        blo_i = plsc.bitcast(bits_lo, jnp.int32)
        bhi_i = plsc.bitcast(bits_hi, jnp.int32)

        def exp_body(t, c):
            sel = jnp.full((16,), 0, jnp.int32) + t
            src = jnp.where(sel < 8, blo_i, bhi_i)
            rowsel = (iota >> 3) + (2 * t - 16 * (t // 8))
            g = plsc.bitcast(src.at[rowsel].get(mode="promise_in_bounds"),
                             jnp.uint32)
            x = (g >> shiftv) & nib
            yb = (x | (x << 7) | (x << 14) | (x << 21)) & lsb4
            off = (_SEQ // 4) * b + 16 * t
            tbuf[pl.ds(off, 16)] = plsc.bitcast(yb, jnp.int32)
            return c

        lax.fori_loop(0, 16, exp_body, jnp.int32(0))
        return carry

    lax.fori_loop(0, _BPW, batch_body, jnp.int32(0))

    nw = _BPW * _SEQ // 4
    pltpu.sync_copy(tbuf, tgt_hbm.at[pl.ds(nw * w, nw)])


@jax.jit
def kernel(u_scale, u_top, u_left):
    mesh = plsc.VectorSubcoreMesh(core_axis_name="c", subcore_axis_name="s",
                                  num_cores=_NC, num_subcores=_NS)
    out_type = jax.ShapeDtypeStruct((_BATCH * _SEQ // 4,), jnp.int32)
    scratch = [
        pltpu.VMEM((16,), jnp.float32),
        pltpu.VMEM((16,), jnp.float32),
        pltpu.VMEM((16,), jnp.float32),
        pltpu.VMEM((_BPW * _SEQ // 4,), jnp.int32),
        pltpu.SemaphoreType.DMA,
    ]
    run = pl.kernel(_body, out_type=out_type, mesh=mesh,
                    compiler_params=pltpu.CompilerParams(
                        needs_layout_passes=False),
                    scratch_types=scratch)
    tgtw = run(u_scale, u_top, u_left)

    by = lax.bitcast_convert_type(tgtw, jnp.uint8).reshape(_BATCH, _SEQ)
    return by == 0, by != 0


# speedup vs baseline: 20.0843x; 1.0266x over previous
import numpy as np
import jax
import jax.numpy as jnp
from jax import lax
from jax.experimental import pallas as pl
from jax.experimental.pallas import tpu as pltpu
from jax.experimental.pallas import tpu_sc as plsc

_BATCH = 64
_H = 32
_W = 32
_SEQ = _H * _W
_NB = 4
_TOTAL = _BATCH * _NB
_SCALE_MIN = 0.15
_SCALE_MAX = 0.2
_ASPECT = 0.75

_NC = 1
_NS = 16
_NW = _NC * _NS
_BPW = _BATCH // _NW


def _sqrt_thresholds(kmax: int = 40) -> np.ndarray:
    ts = []
    for k in range(1, kmax + 1):
        z = np.float32(k * k)
        while True:
            z2 = np.nextafter(z, np.float32(0), dtype=np.float32)
            if np.sqrt(z2, dtype=np.float32) >= np.float32(k):
                z = z2
            else:
                break
        ts.append(z)
    return np.array(ts, dtype=np.float32)


_THRESH = _sqrt_thresholds()


def _body(us_hbm, ut_hbm, ul_hbm, tgt_hbm,
          u_s, u_t, u_l, tbuf, sem):
    w = lax.axis_index("s") * _NC + lax.axis_index("c")
    nblk = _BPW * _NB
    base = w * nblk

    h1 = pltpu.make_async_copy(us_hbm.at[pl.ds(base, nblk)],
                               u_s.at[pl.ds(0, nblk)], sem)
    h2 = pltpu.make_async_copy(ut_hbm.at[pl.ds(base, nblk)],
                               u_t.at[pl.ds(0, nblk)], sem)
    h3 = pltpu.make_async_copy(ul_hbm.at[pl.ds(base, nblk)],
                               u_l.at[pl.ds(0, nblk)], sem)
    h1.start()
    h2.start()
    h3.start()
    h1.wait()
    h2.wait()
    h3.wait()

    us = u_s[...]
    ut = u_t[...]
    ul = u_l[...]

    scales = jnp.float32(_SCALE_MIN) + us * jnp.float32(_SCALE_MAX - _SCALE_MIN)
    area = (scales * jnp.float32(_SEQ)).astype(jnp.int32)
    y = area.astype(jnp.float32) / jnp.float32(_ASPECT)
    hs_raw = jnp.zeros((16,), jnp.int32)
    for k in range(1, 34):
        hs_raw = hs_raw + jnp.where(y >= jnp.float32(_THRESH[k - 1]),
                                    jnp.int32(1), jnp.int32(0))
    hs = jnp.clip(hs_raw, 1, _H)
    ws = jnp.clip(area // hs, 1, _W)
    max_tops = _H - hs + 1
    max_lefts = _W - ws + 1
    tops = (ut * max_tops.astype(jnp.float32)).astype(jnp.int32)
    lefts = (ul * max_lefts.astype(jnp.float32)).astype(jnp.int32)

    ws_u = ws.astype(jnp.uint32)
    ones_w = (jnp.uint32(2) << (ws_u - 1)) - 1
    lef_u = jnp.minimum(lefts, 31).astype(jnp.uint32)
    cm = jnp.where(lefts > 31, jnp.uint32(0), ones_w << lef_u)

    tend = tops + hs
    cm_i = plsc.bitcast(cm, jnp.int32)

    iota = lax.iota(jnp.int32, 16)
    r_lo = iota
    r_hi = iota + 16

    def batch_body(b, carry):
        def blk_body(j, bits):
            bl, bh = bits
            idx = jnp.full((16,), 0, jnp.int32) + (_NB * b + j)
            topb = tops.at[idx].get(mode="promise_in_bounds")
            tendb = tend.at[idx].get(mode="promise_in_bounds")
            cmb = plsc.bitcast(cm_i.at[idx].get(mode="promise_in_bounds"),
                               jnp.uint32)
            z = jnp.uint32(0)
            bl = bl | jnp.where((r_lo >= topb) & (r_lo < tendb), cmb, z)
            bh = bh | jnp.where((r_hi >= topb) & (r_hi < tendb), cmb, z)
            return bl, bh

        zz = jnp.zeros((16,), jnp.uint32)
        bits_lo, bits_hi = lax.fori_loop(0, _NB, blk_body, (zz, zz))
        tbuf[pl.ds(_H * b, 16)] = plsc.bitcast(bits_lo, jnp.int32)
        tbuf[pl.ds(_H * b + 16, 16)] = plsc.bitcast(bits_hi, jnp.int32)
        return carry

    lax.fori_loop(0, _BPW, batch_body, jnp.int32(0))

    nw = _BPW * _H
    pltpu.sync_copy(tbuf, tgt_hbm.at[pl.ds(nw * w, nw)])


@jax.jit
def kernel(u_scale, u_top, u_left):
    mesh = plsc.VectorSubcoreMesh(core_axis_name="c", subcore_axis_name="s",
                                  num_cores=_NC, num_subcores=_NS)
    out_type = jax.ShapeDtypeStruct((_BATCH * _H,), jnp.int32)
    scratch = [
        pltpu.VMEM((16,), jnp.float32),
        pltpu.VMEM((16,), jnp.float32),
        pltpu.VMEM((16,), jnp.float32),
        pltpu.VMEM((_BPW * _H,), jnp.int32),
        pltpu.SemaphoreType.DMA,
    ]
    run = pl.kernel(_body, out_type=out_type, mesh=mesh,
                    compiler_params=pltpu.CompilerParams(
                        needs_layout_passes=False),
                    scratch_types=scratch)
    tgtw = run(u_scale, u_top, u_left)

    bits = tgtw.reshape(_BATCH, _H)
    e = (bits[:, :, None] >> jnp.arange(_W, dtype=jnp.int32)[None, None, :]) & 1
    by = e.reshape(_BATCH, _SEQ)
    return by == 0, by != 0
